# baseline (speedup 1.0000x reference)
import functools

import jax
import jax.numpy as jnp
from jax import lax
from jax.experimental import pallas as pl
from jax.experimental.pallas import tpu as pltpu
from jax.experimental.pallas import tpu_sc as plsc

EMBED_DIM = 64
NUM_CORES = 2
NUM_SUBCORES = 16
NUM_WORKERS = NUM_CORES * NUM_SUBCORES
CHUNK = 160
NBUF = 2


@functools.cache
def _make_gather(num_rows: int):
    assert num_rows % (NUM_WORKERS * CHUNK) == 0
    rows_per_worker = num_rows // NUM_WORKERS
    n_chunks = rows_per_worker // CHUNK
    mesh = plsc.VectorSubcoreMesh(core_axis_name="c", subcore_axis_name="s")

    @functools.partial(
        pl.kernel,
        mesh=mesh,
        out_type=jax.ShapeDtypeStruct((num_rows, EMBED_DIM), jnp.float32),
        scratch_types=[
            pltpu.VMEM((rows_per_worker,), jnp.int32),
            pltpu.VMEM((NBUF, CHUNK, 2 * EMBED_DIM), jnp.float32),
            pltpu.VMEM((NBUF, CHUNK, EMBED_DIM), jnp.float32),
            pltpu.SemaphoreType.DMA((NBUF,)),
            pltpu.SemaphoreType.DMA((NBUF,)),
        ],
    )
    def gather_kernel(idx_hbm, tables_hbm, out_hbm, idx_v, pairs_v, rows_v,
                      gsem, osem):
        wid = lax.axis_index("s") * NUM_CORES + lax.axis_index("c")
        base = wid * rows_per_worker
        pltpu.sync_copy(idx_hbm.at[pl.ds(base, rows_per_worker)], idx_v)

        def gather_start(i, slot):
            pltpu.async_copy(
                tables_hbm.at[idx_v.at[pl.ds(i * CHUNK, CHUNK)]],
                pairs_v.at[slot],
                gsem.at[slot],
            )

        def gather_wait(slot):
            pltpu.make_async_copy(
                tables_hbm.at[idx_v.at[pl.ds(0, CHUNK)]],
                pairs_v.at[slot],
                gsem.at[slot],
            ).wait()

        def out_start(i, slot):
            pltpu.async_copy(
                rows_v.at[slot],
                out_hbm.at[pl.ds(base + i * CHUNK, CHUNK)],
                osem.at[slot],
            )

        def out_wait(slot):
            pltpu.make_async_copy(
                rows_v.at[slot],
                out_hbm.at[pl.ds(base, CHUNK)],
                osem.at[slot],
            ).wait()

        UNROLL = 8

        def compact(slot):
            def body(g, carry):
                for u in range(UNROLL):
                    r = g * UNROLL + u
                    for c in range(EMBED_DIM // 16):
                        rows_v[slot, r, pl.ds(c * 16, 16)] = (
                            pairs_v[slot, r, pl.ds(c * 16, 16)])
                return carry

            lax.fori_loop(0, CHUNK // UNROLL, body, 0)

        gather_start(0, 0)

        def step(i, carry):
            slot = lax.rem(i, NBUF)
            nxt = lax.rem(i + 1, NBUF)

            @pl.when(i + 1 < n_chunks)
            def _():
                gather_start(i + 1, nxt)

            gather_wait(slot)

            @pl.when(i >= NBUF)
            def _():
                out_wait(slot)

            compact(slot)
            out_start(i, slot)
            return carry

        lax.fori_loop(0, n_chunks, step, 0)
        for s in range(NBUF):
            out_wait(s)

    return gather_kernel


def kernel(input_x, table):
    batch, hist = input_x.shape
    vocab = table.shape[0]
    half = vocab // 2
    idx = input_x.reshape(-1).astype(jnp.int32)
    slots = (idx >> 1) + (idx & 1) * half
    tables = jnp.concatenate(
        [table.reshape(half, 2 * EMBED_DIM),
         jnp.roll(table, -1, axis=0).reshape(half, 2 * EMBED_DIM)], axis=0)
    y = _make_gather(idx.shape[0])(slots, tables)
    y = y.reshape(batch, hist, EMBED_DIM)
    return (y, y)


# speedup vs baseline: 1.1420x; 1.1420x over previous
import functools

import jax
import jax.numpy as jnp
from jax import lax
from jax.experimental import pallas as pl
from jax.experimental.pallas import tpu as pltpu
from jax.experimental.pallas import tpu_sc as plsc

EMBED_DIM = 64
NUM_CORES = 2
NUM_SUBCORES = 16
NUM_WORKERS = NUM_CORES * NUM_SUBCORES
CHUNK = 160
NBUF = 2


@functools.cache
def _make_gather(num_rows: int):
    assert num_rows % (NUM_WORKERS * CHUNK) == 0
    rows_per_worker = num_rows // NUM_WORKERS
    n_chunks = rows_per_worker // CHUNK
    mesh = plsc.VectorSubcoreMesh(core_axis_name="c", subcore_axis_name="s")

    @functools.partial(
        pl.kernel,
        mesh=mesh,
        out_type=jax.ShapeDtypeStruct((num_rows, EMBED_DIM), jnp.float32),
        scratch_types=[
            pltpu.VMEM((rows_per_worker,), jnp.int32),
            pltpu.VMEM((NBUF, CHUNK, 2 * EMBED_DIM), jnp.float32),
            pltpu.VMEM((NBUF, CHUNK, EMBED_DIM), jnp.float32),
            pltpu.SemaphoreType.DMA((NBUF,)),
            pltpu.SemaphoreType.DMA((NBUF,)),
        ],
    )
    def gather_kernel(idx_hbm, tables_hbm, out_hbm, idx_v, pairs_v, rows_v,
                      gsem, osem):
        wid = lax.axis_index("s") * NUM_CORES + lax.axis_index("c")
        base = wid * rows_per_worker
        pltpu.sync_copy(idx_hbm.at[pl.ds(base, rows_per_worker)], idx_v)

        def gather_start(i, slot):
            pltpu.async_copy(
                tables_hbm.at[idx_v.at[pl.ds(i * CHUNK, CHUNK)]],
                pairs_v.at[slot],
                gsem.at[slot],
            )

        def gather_wait(slot):
            pltpu.make_async_copy(
                tables_hbm.at[idx_v.at[pl.ds(0, CHUNK)]],
                pairs_v.at[slot],
                gsem.at[slot],
            ).wait()

        def out_start(i, slot):
            pltpu.async_copy(
                rows_v.at[slot],
                out_hbm.at[pl.ds(base + i * CHUNK, CHUNK)],
                osem.at[slot],
            )

        def out_wait(slot):
            pltpu.make_async_copy(
                rows_v.at[slot],
                out_hbm.at[pl.ds(base, CHUNK)],
                osem.at[slot],
            ).wait()

        def compact(slot):
            for r in range(CHUNK):
                for c in range(EMBED_DIM // 16):
                    rows_v[slot, r, pl.ds(c * 16, 16)] = (
                        pairs_v[slot, r, pl.ds(c * 16, 16)])

        gather_start(0, 0)

        def step(i2, carry):
            for u in range(NBUF):
                i = i2 * NBUF + u
                slot = u
                nxt = (u + 1) % NBUF

                @pl.when(i + 1 < n_chunks)
                def _():
                    gather_start(i + 1, nxt)

                gather_wait(slot)

                @pl.when(i >= NBUF)
                def _():
                    out_wait(slot)

                compact(slot)
                out_start(i, slot)
            return carry

        lax.fori_loop(0, n_chunks // NBUF, step, 0)
        for s in range(NBUF):
            out_wait(s)

    return gather_kernel


def kernel(input_x, table):
    batch, hist = input_x.shape
    vocab = table.shape[0]
    half = vocab // 2
    idx = input_x.reshape(-1).astype(jnp.int32)
    slots = (idx >> 1) + (idx & 1) * half
    tables = jnp.concatenate(
        [table.reshape(half, 2 * EMBED_DIM),
         jnp.roll(table, -1, axis=0).reshape(half, 2 * EMBED_DIM)], axis=0)
    y = _make_gather(idx.shape[0])(slots, tables)
    y = y.reshape(batch, hist, EMBED_DIM)
    return (y, y)
